# Initial kernel scaffold; baseline (speedup 1.0000x reference)
import jax
import jax.numpy as jnp
from jax.experimental import pallas as pl


def kernel(x, edge_index, W1, a_src1, a_dst1, b1, W2, a_src2, a_dst2, b2):
    raise NotImplementedError("write your pallas kernel here")


# speedup vs baseline: 42.8033x; 42.8033x over previous
import functools

import jax
import jax.numpy as jnp
from jax import lax
from jax.experimental import pallas as pl
from jax.experimental.pallas import tpu as pltpu
from jax.experimental.pallas import tpu_sc as plsc

N = 10000
E = 320000
D_IN = 128
HID = 16
HEADS = 8
D_OUT = 64

NC = 2
NS = 16
NW = NC * NS
CH = 80
EPW = E // NW
NCHUNK = EPW // CH
NPAD = 10240
ROWS_PER_TILE = NPAD // NS

ROW_BLOCK = 512


def _stage_a_body(x_ref, w_ref, as_ref, ad_ref, h_ref, asrc_ref, adst_ref):
    h = jnp.dot(x_ref[...], w_ref[...], preferred_element_type=jnp.float32)
    h_ref[...] = h
    asrc_ref[...] = jnp.dot(h, as_ref[...], preferred_element_type=jnp.float32)
    adst_ref[...] = jnp.dot(h, ad_ref[...], preferred_element_type=jnp.float32)


def _stage_a(x, W1, As1, Ad1):
    grid = (NPAD // ROW_BLOCK,)
    return pl.pallas_call(
        _stage_a_body,
        grid=grid,
        in_specs=[
            pl.BlockSpec((ROW_BLOCK, D_IN), lambda i: (i, 0)),
            pl.BlockSpec((D_IN, HEADS * HID), lambda i: (0, 0)),
            pl.BlockSpec((HEADS * HID, 16), lambda i: (0, 0)),
            pl.BlockSpec((HEADS * HID, 16), lambda i: (0, 0)),
        ],
        out_specs=[
            pl.BlockSpec((ROW_BLOCK, HEADS * HID), lambda i: (i, 0)),
            pl.BlockSpec((ROW_BLOCK, 16), lambda i: (i, 0)),
            pl.BlockSpec((ROW_BLOCK, 16), lambda i: (i, 0)),
        ],
        out_shape=[
            jax.ShapeDtypeStruct((NPAD, HEADS * HID), jnp.float32),
            jax.ShapeDtypeStruct((NPAD, 16), jnp.float32),
            jax.ShapeDtypeStruct((NPAD, 16), jnp.float32),
        ],
    )(x, W1, As1, Ad1)


def _stage_c_body(accp_ref, denp_ref, b1_ref, r_ref, w2_ref, as2_ref, ad2_ref,
                  h2_ref, asrc2_ref, adst2_ref):
    acc = accp_ref[0] + accp_ref[1]
    den = denp_ref[0] + denp_ref[1]
    den_b = jnp.dot(den, r_ref[...], preferred_element_type=jnp.float32)
    h1 = acc / (den_b + 1e-16) + b1_ref[...]
    h1 = jnp.maximum(h1, 0.01 * h1)
    h2 = jnp.dot(h1, w2_ref[...], preferred_element_type=jnp.float32)
    h2_ref[...] = h2
    asrc2_ref[...] = jnp.dot(h2, as2_ref[...], preferred_element_type=jnp.float32)
    adst2_ref[...] = jnp.dot(h2, ad2_ref[...], preferred_element_type=jnp.float32)


def _stage_c(accp, denp, b1r, R, W2, As2, Ad2):
    grid = (NPAD // ROW_BLOCK,)
    return pl.pallas_call(
        _stage_c_body,
        grid=grid,
        in_specs=[
            pl.BlockSpec((NC, ROW_BLOCK, 128), lambda i: (0, i, 0)),
            pl.BlockSpec((NC, ROW_BLOCK, 16), lambda i: (0, i, 0)),
            pl.BlockSpec((1, 128), lambda i: (0, 0)),
            pl.BlockSpec((16, 128), lambda i: (0, 0)),
            pl.BlockSpec((128, D_OUT), lambda i: (0, 0)),
            pl.BlockSpec((D_OUT, 16), lambda i: (0, 0)),
            pl.BlockSpec((D_OUT, 16), lambda i: (0, 0)),
        ],
        out_specs=[
            pl.BlockSpec((ROW_BLOCK, D_OUT), lambda i: (i, 0)),
            pl.BlockSpec((ROW_BLOCK, 16), lambda i: (i, 0)),
            pl.BlockSpec((ROW_BLOCK, 16), lambda i: (i, 0)),
        ],
        out_shape=[
            jax.ShapeDtypeStruct((NPAD, D_OUT), jnp.float32),
            jax.ShapeDtypeStruct((NPAD, 16), jnp.float32),
            jax.ShapeDtypeStruct((NPAD, 16), jnp.float32),
        ],
    )(accp, denp, b1r, R, W2, As2, Ad2)


def _stage_e_body(accp_ref, denp_ref, b2_ref, r2_ref, out_ref):
    acc = accp_ref[0] + accp_ref[1]
    den = denp_ref[0] + denp_ref[1]
    den_b = jnp.dot(den, r2_ref[...], preferred_element_type=jnp.float32)
    out_ref[...] = acc / (den_b + 1e-16) + b2_ref[...]


def _stage_e(accp, denp, b2r, R2):
    grid = (NPAD // ROW_BLOCK,)
    return pl.pallas_call(
        _stage_e_body,
        grid=grid,
        in_specs=[
            pl.BlockSpec((NC, ROW_BLOCK, D_OUT), lambda i: (0, i, 0)),
            pl.BlockSpec((NC, ROW_BLOCK, 16), lambda i: (0, i, 0)),
            pl.BlockSpec((1, D_OUT), lambda i: (0, 0)),
            pl.BlockSpec((16, D_OUT), lambda i: (0, 0)),
        ],
        out_specs=pl.BlockSpec((ROW_BLOCK, D_OUT), lambda i: (i, 0)),
        out_shape=jax.ShapeDtypeStruct((NPAD, D_OUT), jnp.float32),
    )(accp, denp, b2r, R2)


def _sc_edge_stage(C, n_heads, c_head):
    mesh = plsc.VectorSubcoreMesh(core_axis_name="c", subcore_axis_name="s")
    n_ch = C // 16

    @functools.partial(
        pl.kernel,
        mesh=mesh,
        compiler_params=pltpu.CompilerParams(use_tc_tiling_on_sc=False),
        out_type=[
            jax.ShapeDtypeStruct((NC, NPAD, C), jnp.float32),
            jax.ShapeDtypeStruct((NC, NPAD, 16), jnp.float32),
        ],
        scratch_types=[
            pltpu.VMEM((CH,), jnp.int32),
            pltpu.VMEM((CH,), jnp.int32),
            pltpu.VMEM((CH, 16), jnp.float32),
            pltpu.VMEM((CH, 16), jnp.float32),
            pltpu.VMEM((CH, C), jnp.float32),
            pltpu.VMEM((CH, 16), jnp.float32),
            pltpu.VMEM_SHARED((NPAD, C), jnp.float32),
            pltpu.VMEM_SHARED((NPAD, 16), jnp.float32),
            pltpu.SemaphoreType.DMA,
            pltpu.SemaphoreType.DMA,
            pltpu.SemaphoreType.DMA,
        ],
    )
    def k(src_hbm, dst_hbm, asrc_hbm, adst_hbm, feat_hbm, zc_hbm, z16_hbm,
          accp_hbm, denp_hbm,
          sidx, didx, sbuf, dbuf, fbuf, wbuf, acc_s, den_s,
          sem1, sem2, sem3):
        c = lax.axis_index("c")
        s = lax.axis_index("s")
        wid = c * NS + s
        r0 = s * ROWS_PER_TILE
        pltpu.sync_copy(zc_hbm.at[pl.ds(r0, ROWS_PER_TILE)],
                        acc_s.at[pl.ds(r0, ROWS_PER_TILE)])
        pltpu.sync_copy(z16_hbm.at[pl.ds(r0, ROWS_PER_TILE)],
                        den_s.at[pl.ds(r0, ROWS_PER_TILE)])
        plsc.subcore_barrier()

        lane = lax.iota(jnp.int32, 16)
        e0 = wid * EPW

        def chunk_body(ci, carry):
            base = e0 + ci * CH
            pltpu.sync_copy(src_hbm.at[pl.ds(base, CH)], sidx)
            pltpu.sync_copy(dst_hbm.at[pl.ds(base, CH)], didx)
            cp1 = pltpu.async_copy(asrc_hbm.at[sidx], sbuf, sem1)
            cp2 = pltpu.async_copy(adst_hbm.at[didx], dbuf, sem2)
            cp3 = pltpu.async_copy(feat_hbm.at[sidx], fbuf, sem3)
            cp1.wait()
            cp2.wait()
            cp3.wait()

            def edge_body(e, carry2):
                ee = sbuf[e, :] + dbuf[e, :]
                ee = jnp.maximum(ee, 0.2 * ee)
                w = jnp.exp(ee)
                w = jnp.where(lane < n_heads, w, 0.0)
                wbuf[e, :] = w
                for chi in range(n_ch):
                    hh = (chi * 16) // c_head
                    wh = w[hh]
                    fbuf[e, pl.ds(chi * 16, 16)] = (
                        fbuf[e, pl.ds(chi * 16, 16)] * wh)
                return carry2

            lax.fori_loop(0, CH, edge_body, 0)
            pltpu.sync_copy(wbuf, den_s.at[didx], add=True)
            pltpu.sync_copy(fbuf, acc_s.at[didx], add=True)
            return carry

        lax.fori_loop(0, NCHUNK, chunk_body, 0)
        plsc.subcore_barrier()
        pltpu.sync_copy(acc_s.at[pl.ds(r0, ROWS_PER_TILE)],
                        accp_hbm.at[c, pl.ds(r0, ROWS_PER_TILE)])
        pltpu.sync_copy(den_s.at[pl.ds(r0, ROWS_PER_TILE)],
                        denp_hbm.at[c, pl.ds(r0, ROWS_PER_TILE)])

    return k


def kernel(x, edge_index, W1, a_src1, a_dst1, b1, W2, a_src2, a_dst2, b2):
    src = edge_index[0]
    dst = edge_index[1]

    eye8 = jnp.eye(HEADS, dtype=jnp.float32)
    zpad8 = jnp.zeros((HEADS * HID, 8), jnp.float32)
    As1 = jnp.concatenate(
        [(eye8[:, None, :] * a_src1[:, :, None]).reshape(HEADS * HID, HEADS),
         zpad8], axis=1)
    Ad1 = jnp.concatenate(
        [(eye8[:, None, :] * a_dst1[:, :, None]).reshape(HEADS * HID, HEADS),
         zpad8], axis=1)
    zpad15 = jnp.zeros((D_OUT, 15), jnp.float32)
    As2 = jnp.concatenate([a_src2.T, zpad15], axis=1)
    Ad2 = jnp.concatenate([a_dst2.T, zpad15], axis=1)
    R1 = jnp.concatenate(
        [jnp.kron(eye8, jnp.ones((1, HID), jnp.float32)),
         jnp.zeros((8, HEADS * HID), jnp.float32)], axis=0)
    R2 = jnp.zeros((16, D_OUT), jnp.float32).at[0].set(1.0)
    b1r = b1.reshape(1, HEADS * HID)
    b2r = b2.reshape(1, D_OUT)

    z128 = jnp.zeros((NPAD, HEADS * HID), jnp.float32)
    z64 = jnp.zeros((NPAD, D_OUT), jnp.float32)
    z16 = jnp.zeros((NPAD, 16), jnp.float32)

    xp = jnp.pad(x, ((0, NPAD - N), (0, 0)))
    h1, asrc1, adst1 = _stage_a(xp, W1, As1, Ad1)
    accp1, denp1 = _sc_edge_stage(128, HEADS, HID)(
        src, dst, asrc1, adst1, h1, z128, z16)
    h2, asrc2, adst2 = _stage_c(accp1, denp1, b1r, R1, W2, As2, Ad2)
    accp2, denp2 = _sc_edge_stage(D_OUT, 1, D_OUT)(
        src, dst, asrc2, adst2, h2, z64, z16)
    out = _stage_e(accp2, denp2, b2r, R2)
    return out[:N]
